# Initial kernel scaffold; baseline (speedup 1.0000x reference)
import jax
import jax.numpy as jnp
from jax.experimental import pallas as pl


def kernel(x, w1, b1, w2, b2):
    raise NotImplementedError("write your pallas kernel here")


# speedup vs baseline: 1.9543x; 1.9543x over previous
import jax
import jax.numpy as jnp
from jax.experimental import pallas as pl
from jax.experimental.pallas import tpu as pltpu

_IN = 12
_H = 10
_OUT = 1
_CB1 = _IN
_CW2 = _IN + 1
_CB2 = _IN + 2
_PC = _IN + 3


def _body(x_ref, p_ref, o_ref):
    w1 = p_ref[:, :_IN]
    b1 = p_ref[:, _CB1:_CB1 + 1]
    w2 = p_ref[:, _CW2:_CW2 + 1]
    b2 = p_ref[0:1, _CB2:_CB2 + 1]

    ht = jax.lax.dot_general(
        w1, x_ref[...], (((1,), (1,)), ((), ())),
        preferred_element_type=jnp.float32,
    )
    ht = jnp.tanh(ht + b1)

    o_ref[...] = jnp.sum(ht * w2, axis=0, keepdims=True) + b2


def _pack(w1, b1, w2, b2):
    p = jnp.zeros((_H, _PC), jnp.float32)
    p = p.at[:, :_IN].set(w1.astype(jnp.float32))
    p = p.at[:, _CB1].set(b1.astype(jnp.float32))
    p = p.at[:, _CW2].set(w2[0].astype(jnp.float32))
    p = p.at[0, _CB2].set(b2[0].astype(jnp.float32))
    return p


def kernel(x, w1, b1, w2, b2):
    B = x.shape[0]
    params = _pack(w1, b1, w2, b2)

    tb = 4096
    while B % tb:
        tb //= 2
    x = x.astype(jnp.float32)

    out_t = pl.pallas_call(
        _body,
        out_shape=jax.ShapeDtypeStruct((_OUT, B), jnp.float32),
        grid=(B // tb,),
        in_specs=[
            pl.BlockSpec((tb, _IN), lambda i: (i, 0)),
            pl.BlockSpec((_H, _PC), lambda i: (0, 0)),
        ],
        out_specs=pl.BlockSpec((_OUT, tb), lambda i: (0, i)),
        compiler_params=pltpu.CompilerParams(
            dimension_semantics=("parallel",),
        ),
    )(x, params)

    return out_t.T


# speedup vs baseline: 2.2868x; 1.1701x over previous
import jax
import jax.numpy as jnp
from jax.experimental import pallas as pl
from jax.experimental.pallas import tpu as pltpu

_IN = 12
_H = 10
_OUT = 1
_CB1 = _IN
_CW2 = _IN + 1
_CB2 = _IN + 2
_PC = _IN + 3


_NQ = 4


def _body(*refs):
    x_refs = refs[:_NQ]
    p_ref = refs[_NQ]
    o_ref = refs[_NQ + 1]
    w1 = p_ref[:, :_IN]
    b1 = p_ref[:, _CB1:_CB1 + 1]
    w2 = p_ref[:, _CW2:_CW2 + 1]
    b2 = p_ref[0:1, _CB2:_CB2 + 1]

    tb = x_refs[0].shape[0]
    for k, x_ref in enumerate(x_refs):
        ht = jax.lax.dot_general(
            w1, x_ref[...], (((1,), (1,)), ((), ())),
            preferred_element_type=jnp.float32,
        )
        ht = jnp.tanh(ht + b1)
        o_ref[:, k * tb:(k + 1) * tb] = (
            jnp.sum(ht * w2, axis=0, keepdims=True) + b2)


def _pack(w1, b1, w2, b2):
    p = jnp.zeros((_H, _PC), jnp.float32)
    p = p.at[:, :_IN].set(w1.astype(jnp.float32))
    p = p.at[:, _CB1].set(b1.astype(jnp.float32))
    p = p.at[:, _CW2].set(w2[0].astype(jnp.float32))
    p = p.at[0, _CB2].set(b2[0].astype(jnp.float32))
    return p


def kernel(x, w1, b1, w2, b2):
    B = x.shape[0]
    params = _pack(w1, b1, w2, b2)

    tb = 2048
    while B % (tb * _NQ):
        tb //= 2
    x = x.astype(jnp.float32)
    if tb < 8:
        return _kernel_single(x, params, B)

    out_t = pl.pallas_call(
        _body,
        out_shape=jax.ShapeDtypeStruct((_OUT, B), jnp.float32),
        grid=(B // (tb * _NQ),),
        in_specs=[
            pl.BlockSpec((tb, _IN), (lambda i, k=k: (_NQ * i + k, 0)))
            for k in range(_NQ)
        ] + [
            pl.BlockSpec((_H, _PC), lambda i: (0, 0)),
        ],
        out_specs=pl.BlockSpec((_OUT, tb * _NQ), lambda i: (0, i)),
        compiler_params=pltpu.CompilerParams(
            dimension_semantics=("parallel",),
        ),
    )(*([x] * _NQ), params)

    return out_t.T


def _single_body(x_ref, p_ref, o_ref):
    w1 = p_ref[:, :_IN]
    b1 = p_ref[:, _CB1:_CB1 + 1]
    w2 = p_ref[:, _CW2:_CW2 + 1]
    b2 = p_ref[0:1, _CB2:_CB2 + 1]
    ht = jax.lax.dot_general(
        w1, x_ref[...], (((1,), (1,)), ((), ())),
        preferred_element_type=jnp.float32,
    )
    ht = jnp.tanh(ht + b1)
    o_ref[...] = jnp.sum(ht * w2, axis=0, keepdims=True) + b2


def _kernel_single(x, params, B):
    tb = B
    for cand in (512, 256, 128, 64, 32, 16, 8, 4, 2, 1):
        if B % cand == 0:
            tb = cand
            break
    out_t = pl.pallas_call(
        _single_body,
        out_shape=jax.ShapeDtypeStruct((_OUT, B), jnp.float32),
        grid=(B // tb,),
        in_specs=[
            pl.BlockSpec((tb, _IN), lambda i: (i, 0)),
            pl.BlockSpec((_H, _PC), lambda i: (0, 0)),
        ],
        out_specs=pl.BlockSpec((_OUT, tb), lambda i: (0, i)),
        compiler_params=pltpu.CompilerParams(
            dimension_semantics=("parallel",),
        ),
    )(x, params)
    return out_t.T


# speedup vs baseline: 2.4526x; 1.0725x over previous
import jax
import jax.numpy as jnp
from jax.experimental import pallas as pl
from jax.experimental.pallas import tpu as pltpu

_IN = 12
_H = 10
_OUT = 1
_CB1 = _IN
_CW2 = _IN + 1
_CB2 = _IN + 2
_PC = _IN + 3


_NQ = 8


def _body(*refs):
    x_refs = refs[:_NQ]
    p_ref = refs[_NQ]
    o_ref = refs[_NQ + 1]
    w1 = p_ref[:, :_IN]
    b1 = p_ref[:, _CB1:_CB1 + 1]
    w2 = p_ref[:, _CW2:_CW2 + 1]
    b2 = p_ref[0:1, _CB2:_CB2 + 1]

    tb = x_refs[0].shape[0]
    for k, x_ref in enumerate(x_refs):
        ht = jax.lax.dot_general(
            w1, x_ref[...], (((1,), (1,)), ((), ())),
            preferred_element_type=jnp.float32,
        )
        ht = jnp.tanh(ht + b1)
        o_ref[:, k * tb:(k + 1) * tb] = (
            jnp.sum(ht * w2, axis=0, keepdims=True) + b2)


def _pack(w1, b1, w2, b2):
    p = jnp.zeros((_H, _PC), jnp.float32)
    p = p.at[:, :_IN].set(w1.astype(jnp.float32))
    p = p.at[:, _CB1].set(b1.astype(jnp.float32))
    p = p.at[:, _CW2].set(w2[0].astype(jnp.float32))
    p = p.at[0, _CB2].set(b2[0].astype(jnp.float32))
    return p


def kernel(x, w1, b1, w2, b2):
    B = x.shape[0]
    params = _pack(w1, b1, w2, b2)

    tb = 2048
    while B % (tb * _NQ):
        tb //= 2
    x = x.astype(jnp.float32)
    if tb < 8:
        return _kernel_single(x, params, B)

    out_t = pl.pallas_call(
        _body,
        out_shape=jax.ShapeDtypeStruct((_OUT, B), jnp.float32),
        grid=(B // (tb * _NQ),),
        in_specs=[
            pl.BlockSpec((tb, _IN), (lambda i, k=k: (_NQ * i + k, 0)))
            for k in range(_NQ)
        ] + [
            pl.BlockSpec((_H, _PC), lambda i: (0, 0)),
        ],
        out_specs=pl.BlockSpec((_OUT, tb * _NQ), lambda i: (0, i)),
        compiler_params=pltpu.CompilerParams(
            dimension_semantics=("parallel",),
        ),
    )(*([x] * _NQ), params)

    return out_t.T


def _single_body(x_ref, p_ref, o_ref):
    w1 = p_ref[:, :_IN]
    b1 = p_ref[:, _CB1:_CB1 + 1]
    w2 = p_ref[:, _CW2:_CW2 + 1]
    b2 = p_ref[0:1, _CB2:_CB2 + 1]
    ht = jax.lax.dot_general(
        w1, x_ref[...], (((1,), (1,)), ((), ())),
        preferred_element_type=jnp.float32,
    )
    ht = jnp.tanh(ht + b1)
    o_ref[...] = jnp.sum(ht * w2, axis=0, keepdims=True) + b2


def _kernel_single(x, params, B):
    tb = B
    for cand in (512, 256, 128, 64, 32, 16, 8, 4, 2, 1):
        if B % cand == 0:
            tb = cand
            break
    out_t = pl.pallas_call(
        _single_body,
        out_shape=jax.ShapeDtypeStruct((_OUT, B), jnp.float32),
        grid=(B // tb,),
        in_specs=[
            pl.BlockSpec((tb, _IN), lambda i: (i, 0)),
            pl.BlockSpec((_H, _PC), lambda i: (0, 0)),
        ],
        out_specs=pl.BlockSpec((_OUT, tb), lambda i: (0, i)),
        compiler_params=pltpu.CompilerParams(
            dimension_semantics=("parallel",),
        ),
    )(x, params)
    return out_t.T
